# baseline (speedup 1.0000x reference)
import functools

import jax
import jax.numpy as jnp
from jax import lax
from jax.experimental import pallas as pl
from jax.experimental.pallas import tpu as pltpu
from jax.experimental.pallas import tpu_sc as plsc

N = 10000
E = 320000
HD1 = 128
HD2 = 64

NC = 2
NS = 16
NW = NC * NS
EPW = E // NW
CHUNK = 80
NCHUNK = EPW // CHUNK
RPT = 624
RTAIL = N - NS * RPT

ROW_BLK = 1000


def _att_body(x_ref, w1_ref, w2_ref, wg_ref, out_ref):
    x = x_ref[...]
    a = jnp.maximum(
        lax.dot_general(x, w1_ref[...], (((1,), (0,)), ((), ())),
                        preferred_element_type=jnp.float32), 0.0)
    s = lax.dot_general(a, w2_ref[...], (((1,), (0,)), ((), ())),
                        preferred_element_type=jnp.float32)
    s = s - jnp.max(s, axis=1, keepdims=True)
    e = jnp.exp(s)
    z = x * (e / jnp.sum(e, axis=1, keepdims=True))
    out_ref[...] = lax.dot_general(z, wg_ref[...], (((1,), (0,)), ((), ())),
                                   preferred_element_type=jnp.float32)


def _attention_support(x, w1, w2, wg):
    return pl.pallas_call(
        _att_body,
        grid=(N // ROW_BLK,),
        in_specs=[
            pl.BlockSpec((ROW_BLK, HD2), lambda i: (i, 0)),
            pl.BlockSpec((HD2, HD1), lambda i: (0, 0)),
            pl.BlockSpec((HD1, HD2), lambda i: (0, 0)),
            pl.BlockSpec((HD2, HD1), lambda i: (0, 0)),
        ],
        out_specs=pl.BlockSpec((ROW_BLK, HD1), lambda i: (i, 0)),
        out_shape=jax.ShapeDtypeStruct((N, HD1), jnp.float32),
    )(x, w1, w2, wg)


def _segsum_body(support_hbm, src_hbm, dst_hbm, zeros_hbm, out_hbm,
                 src_v, dst_v, rows_a, rows_b, agg_s, sem_ga, sem_gb):
    cid = lax.axis_index("c")
    sid = lax.axis_index("s")
    wid = sid * NC + cid

    pltpu.sync_copy(zeros_hbm.at[pl.ds(sid * RPT, RPT)],
                    agg_s.at[pl.ds(sid * RPT, RPT)])

    @pl.when(sid == NS - 1)
    def _zero_tail():
        pltpu.sync_copy(zeros_hbm.at[pl.ds(NS * RPT, RTAIL)],
                        agg_s.at[pl.ds(NS * RPT, RTAIL)])

    pltpu.sync_copy(src_hbm.at[wid], src_v)
    pltpu.sync_copy(dst_hbm.at[wid], dst_v)
    plsc.subcore_barrier()

    def _gather_desc(j, buf, sem):
        return pltpu.make_async_copy(
            support_hbm.at[src_v.at[pl.ds(j * CHUNK, CHUNK)]], buf, sem)

    def _gather(j, buf, sem):
        _gather_desc(j, buf, sem).start()

    def _gather_wait(j, buf, sem):
        _gather_desc(j, buf, sem).wait()

    _gather(0, rows_a, sem_ga)

    @pl.loop(0, (NCHUNK - 1) // 2)
    def _edge_pair(i):
        j = 2 * i
        _gather(j + 1, rows_b, sem_gb)
        _gather_wait(j, rows_a, sem_ga)
        pltpu.sync_copy(rows_a, agg_s.at[dst_v.at[j]], add=True)
        _gather(j + 2, rows_a, sem_ga)
        _gather_wait(j + 1, rows_b, sem_gb)
        pltpu.sync_copy(rows_b, agg_s.at[dst_v.at[j + 1]], add=True)

    _gather_wait(NCHUNK - 1, rows_a, sem_ga)
    pltpu.sync_copy(rows_a, agg_s.at[dst_v.at[NCHUNK - 1]], add=True)

    plsc.subcore_barrier()
    pltpu.sync_copy(agg_s.at[pl.ds(sid * RPT, RPT)],
                    out_hbm.at[cid, pl.ds(sid * RPT, RPT)])

    @pl.when(sid == NS - 1)
    def _out_tail():
        pltpu.sync_copy(agg_s.at[pl.ds(NS * RPT, RTAIL)],
                        out_hbm.at[cid, pl.ds(NS * RPT, RTAIL)])


@functools.cache
def _build_segsum():
    return pl.kernel(
        _segsum_body,
        mesh=plsc.VectorSubcoreMesh(core_axis_name="c", subcore_axis_name="s"),
        out_type=jax.ShapeDtypeStruct((NC, N, HD1), jnp.float32),
        scratch_types=[
            pltpu.VMEM((EPW,), jnp.int32),
            pltpu.VMEM((NCHUNK, CHUNK), jnp.int32),
            pltpu.VMEM((CHUNK, HD1), jnp.float32),
            pltpu.VMEM((CHUNK, HD1), jnp.float32),
            pltpu.VMEM_SHARED((N, HD1), jnp.float32),
            pltpu.SemaphoreType.DMA,
            pltpu.SemaphoreType.DMA,
        ],
    )


def _bn_body(p_ref, g_ref, b_ref, out_ref):
    h = jnp.maximum(p_ref[0] + p_ref[1], 0.0)
    mean = jnp.mean(h, axis=0, keepdims=True)
    var = jnp.mean(jnp.square(h - mean), axis=0, keepdims=True)
    inv = lax.rsqrt(var + 1e-5)
    out_ref[...] = (h - mean) * inv * g_ref[...] + b_ref[...]


def _batchnorm(partials, gamma, beta):
    return pl.pallas_call(
        _bn_body,
        in_specs=[
            pl.BlockSpec((NC, N, HD1), lambda: (0, 0, 0)),
            pl.BlockSpec((1, HD1), lambda: (0, 0)),
            pl.BlockSpec((1, HD1), lambda: (0, 0)),
        ],
        out_specs=pl.BlockSpec((N, HD1), lambda: (0, 0)),
        out_shape=jax.ShapeDtypeStruct((N, HD1), jnp.float32),
    )(partials, gamma.reshape(1, HD1), beta.reshape(1, HD1))


def _mm_body(a_ref, b_ref, o_ref):
    o_ref[...] = lax.dot_general(a_ref[...], b_ref[...],
                                 (((1,), (1,)), ((), ())),
                                 preferred_element_type=jnp.float32)


MM_BLK = 400


def _gram(h):
    return pl.pallas_call(
        _mm_body,
        grid=(N // MM_BLK,),
        in_specs=[
            pl.BlockSpec((MM_BLK, HD1), lambda i: (i, 0)),
            pl.BlockSpec((N, HD1), lambda i: (0, 0)),
        ],
        out_specs=pl.BlockSpec((MM_BLK, N), lambda i: (i, 0)),
        out_shape=jax.ShapeDtypeStruct((N, N), jnp.float32),
    )(h, h)


def kernel(encoder_layer_2, adj, W_att1, W_att2, W_gcn, bn_gamma, bn_beta):
    support = _attention_support(encoder_layer_2, W_att1, W_att2, W_gcn)
    src = adj[1].astype(jnp.int32).reshape(NW, EPW)
    dst = adj[0].astype(jnp.int32).reshape(NW, NCHUNK, CHUNK)
    zeros = jnp.zeros((N, HD1), jnp.float32)
    partials = _build_segsum()(support, src, dst, zeros)
    h_bn = _batchnorm(partials, bn_gamma, bn_beta)
    return _gram(h_bn)


# speedup vs baseline: 1.0187x; 1.0187x over previous
import functools

import jax
import jax.numpy as jnp
from jax import lax
from jax.experimental import pallas as pl
from jax.experimental.pallas import tpu as pltpu
from jax.experimental.pallas import tpu_sc as plsc

N = 10000
E = 320000
HD1 = 128
HD2 = 64

NC = 2
NS = 16
NW = NC * NS
EPW = E // NW
CHUNK = 80
NCHUNK = EPW // CHUNK
RPT = 624
RTAIL = N - NS * RPT

ROW_BLK = 1000


def _att_body(x_ref, w1_ref, w2_ref, wg_ref, out_ref):
    x = x_ref[...]
    a = jnp.maximum(
        lax.dot_general(x, w1_ref[...], (((1,), (0,)), ((), ())),
                        preferred_element_type=jnp.float32), 0.0)
    s = lax.dot_general(a, w2_ref[...], (((1,), (0,)), ((), ())),
                        preferred_element_type=jnp.float32)
    s = s - jnp.max(s, axis=1, keepdims=True)
    e = jnp.exp(s)
    z = x * (e / jnp.sum(e, axis=1, keepdims=True))
    out_ref[...] = lax.dot_general(z, wg_ref[...], (((1,), (0,)), ((), ())),
                                   preferred_element_type=jnp.float32)


def _attention_support(x, w1, w2, wg):
    return pl.pallas_call(
        _att_body,
        grid=(N // ROW_BLK,),
        in_specs=[
            pl.BlockSpec((ROW_BLK, HD2), lambda i: (i, 0)),
            pl.BlockSpec((HD2, HD1), lambda i: (0, 0)),
            pl.BlockSpec((HD1, HD2), lambda i: (0, 0)),
            pl.BlockSpec((HD2, HD1), lambda i: (0, 0)),
        ],
        out_specs=pl.BlockSpec((ROW_BLK, HD1), lambda i: (i, 0)),
        out_shape=jax.ShapeDtypeStruct((N, HD1), jnp.float32),
    )(x, w1, w2, wg)


def _segsum_body(support_hbm, src_hbm, dst_hbm, zeros_hbm, out_hbm,
                 src_v, dst_v, rows_a, rows_b, agg_s, sem_ga, sem_gb):
    cid = lax.axis_index("c")
    sid = lax.axis_index("s")
    wid = sid * NC + cid

    pltpu.sync_copy(zeros_hbm.at[pl.ds(sid * RPT, RPT)],
                    agg_s.at[pl.ds(sid * RPT, RPT)])

    @pl.when(sid == NS - 1)
    def _zero_tail():
        pltpu.sync_copy(zeros_hbm.at[pl.ds(NS * RPT, RTAIL)],
                        agg_s.at[pl.ds(NS * RPT, RTAIL)])

    pltpu.sync_copy(src_hbm.at[wid], src_v)
    pltpu.sync_copy(dst_hbm.at[wid], dst_v)
    plsc.subcore_barrier()

    def _gather_desc(j, buf, sem):
        return pltpu.make_async_copy(
            support_hbm.at[src_v.at[pl.ds(j * CHUNK, CHUNK)]], buf, sem)

    def _gather(j, buf, sem):
        _gather_desc(j, buf, sem).start()

    def _gather_wait(j, buf, sem):
        _gather_desc(j, buf, sem).wait()

    _gather(0, rows_a, sem_ga)

    @pl.loop(0, (NCHUNK - 1) // 2)
    def _edge_pair(i):
        j = 2 * i
        _gather(j + 1, rows_b, sem_gb)
        _gather_wait(j, rows_a, sem_ga)
        pltpu.sync_copy(rows_a, agg_s.at[dst_v.at[j]], add=True)
        _gather(j + 2, rows_a, sem_ga)
        _gather_wait(j + 1, rows_b, sem_gb)
        pltpu.sync_copy(rows_b, agg_s.at[dst_v.at[j + 1]], add=True)

    _gather_wait(NCHUNK - 1, rows_a, sem_ga)
    pltpu.sync_copy(rows_a, agg_s.at[dst_v.at[NCHUNK - 1]], add=True)

    plsc.subcore_barrier()
    pltpu.sync_copy(agg_s.at[pl.ds(sid * RPT, RPT)],
                    out_hbm.at[cid, pl.ds(sid * RPT, RPT)])

    @pl.when(sid == NS - 1)
    def _out_tail():
        pltpu.sync_copy(agg_s.at[pl.ds(NS * RPT, RTAIL)],
                        out_hbm.at[cid, pl.ds(NS * RPT, RTAIL)])


@functools.cache
def _build_segsum():
    return pl.kernel(
        _segsum_body,
        mesh=plsc.VectorSubcoreMesh(core_axis_name="c", subcore_axis_name="s"),
        out_type=jax.ShapeDtypeStruct((NC, N, HD1), jnp.float32),
        scratch_types=[
            pltpu.VMEM((EPW,), jnp.int32),
            pltpu.VMEM((NCHUNK, CHUNK), jnp.int32),
            pltpu.VMEM((CHUNK, HD1), jnp.float32),
            pltpu.VMEM((CHUNK, HD1), jnp.float32),
            pltpu.VMEM_SHARED((N, HD1), jnp.float32),
            pltpu.SemaphoreType.DMA,
            pltpu.SemaphoreType.DMA,
        ],
    )


MM_BLK = 200


def _bn_gram_body(p_ref, g_ref, b_ref, o_ref, h_scr):
    i = pl.program_id(0)

    @pl.when(i == 0)
    def _stats():
        h = jnp.maximum(p_ref[0] + p_ref[1], 0.0)
        mean = jnp.mean(h, axis=0, keepdims=True)
        var = jnp.mean(jnp.square(h - mean), axis=0, keepdims=True)
        inv = lax.rsqrt(var + 1e-5)
        h_scr[...] = (h - mean) * inv * g_ref[...] + b_ref[...]

    a = h_scr[pl.ds(i * MM_BLK, MM_BLK), :]
    o_ref[...] = lax.dot_general(a, h_scr[...], (((1,), (1,)), ((), ())),
                                 preferred_element_type=jnp.float32)


def _bn_gram(partials, gamma, beta):
    return pl.pallas_call(
        _bn_gram_body,
        grid=(N // MM_BLK,),
        in_specs=[
            pl.BlockSpec((NC, N, HD1), lambda i: (0, 0, 0)),
            pl.BlockSpec((1, HD1), lambda i: (0, 0)),
            pl.BlockSpec((1, HD1), lambda i: (0, 0)),
        ],
        out_specs=pl.BlockSpec((MM_BLK, N), lambda i: (i, 0)),
        out_shape=jax.ShapeDtypeStruct((N, N), jnp.float32),
        scratch_shapes=[pltpu.VMEM((N, HD1), jnp.float32)],
    )(partials, gamma.reshape(1, HD1), beta.reshape(1, HD1))


def kernel(encoder_layer_2, adj, W_att1, W_att2, W_gcn, bn_gamma, bn_beta):
    support = _attention_support(encoder_layer_2, W_att1, W_att2, W_gcn)
    src = adj[1].astype(jnp.int32).reshape(NW, EPW)
    dst = adj[0].astype(jnp.int32).reshape(NW, NCHUNK, CHUNK)
    zeros = jnp.zeros((N, HD1), jnp.float32)
    partials = _build_segsum()(support, src, dst, zeros)
    return _bn_gram(partials, bn_gamma, bn_beta)


# speedup vs baseline: 1.0246x; 1.0058x over previous
import functools

import jax
import jax.numpy as jnp
from jax import lax
from jax.experimental import pallas as pl
from jax.experimental.pallas import tpu as pltpu
from jax.experimental.pallas import tpu_sc as plsc

N = 10000
E = 320000
HD1 = 128
HD2 = 64

NC = 2
NS = 16
NW = NC * NS
EPW = E // NW
CHUNK = 80
NCHUNK = EPW // CHUNK
RPT = 624
RTAIL = N - NS * RPT

ROW_BLK = 1000


def _att_body(x_ref, w1_ref, w2_ref, wg_ref, out_ref):
    x = x_ref[...]
    a = jnp.maximum(
        lax.dot_general(x, w1_ref[...], (((1,), (0,)), ((), ())),
                        preferred_element_type=jnp.float32), 0.0)
    s = lax.dot_general(a, w2_ref[...], (((1,), (0,)), ((), ())),
                        preferred_element_type=jnp.float32)
    s = s - jnp.max(s, axis=1, keepdims=True)
    e = jnp.exp(s)
    z = x * (e / jnp.sum(e, axis=1, keepdims=True))
    out_ref[...] = lax.dot_general(z, wg_ref[...], (((1,), (0,)), ((), ())),
                                   preferred_element_type=jnp.float32)


def _attention_support(x, w1, w2, wg):
    return pl.pallas_call(
        _att_body,
        grid=(N // ROW_BLK,),
        in_specs=[
            pl.BlockSpec((ROW_BLK, HD2), lambda i: (i, 0)),
            pl.BlockSpec((HD2, HD1), lambda i: (0, 0)),
            pl.BlockSpec((HD1, HD2), lambda i: (0, 0)),
            pl.BlockSpec((HD2, HD1), lambda i: (0, 0)),
        ],
        out_specs=pl.BlockSpec((ROW_BLK, HD1), lambda i: (i, 0)),
        out_shape=jax.ShapeDtypeStruct((N, HD1), jnp.float32),
    )(x, w1, w2, wg)


def _segsum_body(support_hbm, src_hbm, dst_hbm, zeros_hbm, out_hbm,
                 src_v, dst_v, rows_a, rows_b, agg_s, sem_ga, sem_gb):
    cid = lax.axis_index("c")
    sid = lax.axis_index("s")
    wid = sid * NC + cid

    def _gather_desc(j, buf, sem):
        return pltpu.make_async_copy(
            support_hbm.at[src_v.at[pl.ds(j * CHUNK, CHUNK)]], buf, sem)

    def _gather(j, buf, sem):
        _gather_desc(j, buf, sem).start()

    def _gather_wait(j, buf, sem):
        _gather_desc(j, buf, sem).wait()

    pltpu.sync_copy(src_hbm.at[wid], src_v)
    _gather(0, rows_a, sem_ga)

    pltpu.sync_copy(zeros_hbm.at[pl.ds(sid * RPT, RPT)],
                    agg_s.at[pl.ds(sid * RPT, RPT)])

    @pl.when(sid == NS - 1)
    def _zero_tail():
        pltpu.sync_copy(zeros_hbm.at[pl.ds(NS * RPT, RTAIL)],
                        agg_s.at[pl.ds(NS * RPT, RTAIL)])

    pltpu.sync_copy(dst_hbm.at[wid], dst_v)
    plsc.subcore_barrier()


    @pl.loop(0, (NCHUNK - 1) // 2)
    def _edge_pair(i):
        j = 2 * i
        _gather(j + 1, rows_b, sem_gb)
        _gather_wait(j, rows_a, sem_ga)
        pltpu.sync_copy(rows_a, agg_s.at[dst_v.at[j]], add=True)
        _gather(j + 2, rows_a, sem_ga)
        _gather_wait(j + 1, rows_b, sem_gb)
        pltpu.sync_copy(rows_b, agg_s.at[dst_v.at[j + 1]], add=True)

    _gather_wait(NCHUNK - 1, rows_a, sem_ga)
    pltpu.sync_copy(rows_a, agg_s.at[dst_v.at[NCHUNK - 1]], add=True)

    plsc.subcore_barrier()
    pltpu.sync_copy(agg_s.at[pl.ds(sid * RPT, RPT)],
                    out_hbm.at[cid, pl.ds(sid * RPT, RPT)])

    @pl.when(sid == NS - 1)
    def _out_tail():
        pltpu.sync_copy(agg_s.at[pl.ds(NS * RPT, RTAIL)],
                        out_hbm.at[cid, pl.ds(NS * RPT, RTAIL)])


@functools.cache
def _build_segsum():
    return pl.kernel(
        _segsum_body,
        mesh=plsc.VectorSubcoreMesh(core_axis_name="c", subcore_axis_name="s"),
        out_type=jax.ShapeDtypeStruct((NC, N, HD1), jnp.float32),
        scratch_types=[
            pltpu.VMEM((EPW,), jnp.int32),
            pltpu.VMEM((NCHUNK, CHUNK), jnp.int32),
            pltpu.VMEM((CHUNK, HD1), jnp.float32),
            pltpu.VMEM((CHUNK, HD1), jnp.float32),
            pltpu.VMEM_SHARED((N, HD1), jnp.float32),
            pltpu.SemaphoreType.DMA,
            pltpu.SemaphoreType.DMA,
        ],
    )


MM_BLK = 200


def _bn_gram_body(p_ref, g_ref, b_ref, o_ref, h_scr):
    i = pl.program_id(0)

    @pl.when(i == 0)
    def _stats():
        h = jnp.maximum(p_ref[0] + p_ref[1], 0.0)
        mean = jnp.mean(h, axis=0, keepdims=True)
        var = jnp.mean(jnp.square(h - mean), axis=0, keepdims=True)
        inv = lax.rsqrt(var + 1e-5)
        h_scr[...] = (h - mean) * inv * g_ref[...] + b_ref[...]

    a = h_scr[pl.ds(i * MM_BLK, MM_BLK), :]
    o_ref[...] = lax.dot_general(a, h_scr[...], (((1,), (1,)), ((), ())),
                                 preferred_element_type=jnp.float32)


def _bn_gram(partials, gamma, beta):
    return pl.pallas_call(
        _bn_gram_body,
        grid=(N // MM_BLK,),
        in_specs=[
            pl.BlockSpec((NC, N, HD1), lambda i: (0, 0, 0)),
            pl.BlockSpec((1, HD1), lambda i: (0, 0)),
            pl.BlockSpec((1, HD1), lambda i: (0, 0)),
        ],
        out_specs=pl.BlockSpec((MM_BLK, N), lambda i: (i, 0)),
        out_shape=jax.ShapeDtypeStruct((N, N), jnp.float32),
        scratch_shapes=[pltpu.VMEM((N, HD1), jnp.float32)],
    )(partials, gamma.reshape(1, HD1), beta.reshape(1, HD1))


def kernel(encoder_layer_2, adj, W_att1, W_att2, W_gcn, bn_gamma, bn_beta):
    support = _attention_support(encoder_layer_2, W_att1, W_att2, W_gcn)
    src = adj[1].astype(jnp.int32).reshape(NW, EPW)
    dst = adj[0].astype(jnp.int32).reshape(NW, NCHUNK, CHUNK)
    zeros = jnp.zeros((N, HD1), jnp.float32)
    partials = _build_segsum()(support, src, dst, zeros)
    return _bn_gram(partials, bn_gamma, bn_beta)


# speedup vs baseline: 1.0261x; 1.0015x over previous
import functools

import jax
import jax.numpy as jnp
from jax import lax
from jax.experimental import pallas as pl
from jax.experimental.pallas import tpu as pltpu
from jax.experimental.pallas import tpu_sc as plsc

N = 10000
E = 320000
HD1 = 128
HD2 = 64

NC = 2
NS = 16
NW = NC * NS
EPW = E // NW
CHUNK = 80
NCHUNK = EPW // CHUNK
RPT = 624
RTAIL = N - NS * RPT

ROW_BLK = 1000


def _att_body(x_ref, w1_ref, w2_ref, wg_ref, out_ref):
    x = x_ref[...]
    a = jnp.maximum(
        lax.dot_general(x, w1_ref[...], (((1,), (0,)), ((), ())),
                        preferred_element_type=jnp.float32), 0.0)
    s = lax.dot_general(a, w2_ref[...], (((1,), (0,)), ((), ())),
                        preferred_element_type=jnp.float32)
    s = s - jnp.max(s, axis=1, keepdims=True)
    e = jnp.exp(s)
    z = x * (e / jnp.sum(e, axis=1, keepdims=True))
    out_ref[...] = lax.dot_general(z, wg_ref[...], (((1,), (0,)), ((), ())),
                                   preferred_element_type=jnp.float32)


def _attention_support(x, w1, w2, wg):
    return pl.pallas_call(
        _att_body,
        grid=(N // ROW_BLK,),
        in_specs=[
            pl.BlockSpec((ROW_BLK, HD2), lambda i: (i, 0)),
            pl.BlockSpec((HD2, HD1), lambda i: (0, 0)),
            pl.BlockSpec((HD1, HD2), lambda i: (0, 0)),
            pl.BlockSpec((HD2, HD1), lambda i: (0, 0)),
        ],
        out_specs=pl.BlockSpec((ROW_BLK, HD1), lambda i: (i, 0)),
        out_shape=jax.ShapeDtypeStruct((N, HD1), jnp.float32),
    )(x, w1, w2, wg)


def _segsum_body(support_hbm, src_hbm, dst_hbm, zeros_hbm, out_hbm,
                 src_v, dst_v, rows_a, rows_b, agg_s, sem_ga, sem_gb):
    cid = lax.axis_index("c")
    sid = lax.axis_index("s")
    wid = sid * NC + cid

    def _gather_desc(j, buf, sem):
        return pltpu.make_async_copy(
            support_hbm.at[src_v.at[pl.ds(j * CHUNK, CHUNK)]], buf, sem)

    def _gather(j, buf, sem):
        _gather_desc(j, buf, sem).start()

    def _gather_wait(j, buf, sem):
        _gather_desc(j, buf, sem).wait()

    pltpu.sync_copy(src_hbm.at[wid], src_v)
    _gather(0, rows_a, sem_ga)
    _gather(1, rows_b, sem_gb)

    pltpu.sync_copy(zeros_hbm.at[pl.ds(sid * RPT, RPT)],
                    agg_s.at[pl.ds(sid * RPT, RPT)])

    @pl.when(sid == NS - 1)
    def _zero_tail():
        pltpu.sync_copy(zeros_hbm.at[pl.ds(NS * RPT, RTAIL)],
                        agg_s.at[pl.ds(NS * RPT, RTAIL)])

    pltpu.sync_copy(dst_hbm.at[wid], dst_v)
    plsc.subcore_barrier()


    @pl.loop(0, (NCHUNK - 3) // 2)
    def _edge_pair(i):
        j = 2 * i
        _gather_wait(j, rows_a, sem_ga)
        pltpu.sync_copy(rows_a, agg_s.at[dst_v.at[j]], add=True)
        _gather(j + 2, rows_a, sem_ga)
        _gather_wait(j + 1, rows_b, sem_gb)
        pltpu.sync_copy(rows_b, agg_s.at[dst_v.at[j + 1]], add=True)
        _gather(j + 3, rows_b, sem_gb)

    _gather_wait(NCHUNK - 3, rows_a, sem_ga)
    pltpu.sync_copy(rows_a, agg_s.at[dst_v.at[NCHUNK - 3]], add=True)
    _gather(NCHUNK - 1, rows_a, sem_ga)
    _gather_wait(NCHUNK - 2, rows_b, sem_gb)
    pltpu.sync_copy(rows_b, agg_s.at[dst_v.at[NCHUNK - 2]], add=True)
    _gather_wait(NCHUNK - 1, rows_a, sem_ga)
    pltpu.sync_copy(rows_a, agg_s.at[dst_v.at[NCHUNK - 1]], add=True)

    plsc.subcore_barrier()
    pltpu.sync_copy(agg_s.at[pl.ds(sid * RPT, RPT)],
                    out_hbm.at[cid, pl.ds(sid * RPT, RPT)])

    @pl.when(sid == NS - 1)
    def _out_tail():
        pltpu.sync_copy(agg_s.at[pl.ds(NS * RPT, RTAIL)],
                        out_hbm.at[cid, pl.ds(NS * RPT, RTAIL)])


@functools.cache
def _build_segsum():
    return pl.kernel(
        _segsum_body,
        mesh=plsc.VectorSubcoreMesh(core_axis_name="c", subcore_axis_name="s"),
        out_type=jax.ShapeDtypeStruct((NC, N, HD1), jnp.float32),
        scratch_types=[
            pltpu.VMEM((EPW,), jnp.int32),
            pltpu.VMEM((NCHUNK, CHUNK), jnp.int32),
            pltpu.VMEM((CHUNK, HD1), jnp.float32),
            pltpu.VMEM((CHUNK, HD1), jnp.float32),
            pltpu.VMEM_SHARED((N, HD1), jnp.float32),
            pltpu.SemaphoreType.DMA,
            pltpu.SemaphoreType.DMA,
        ],
    )


MM_BLK = 200


def _bn_gram_body(p_ref, g_ref, b_ref, o_ref, h_scr):
    i = pl.program_id(0)

    @pl.when(i == 0)
    def _stats():
        h = jnp.maximum(p_ref[0] + p_ref[1], 0.0)
        mean = jnp.mean(h, axis=0, keepdims=True)
        var = jnp.mean(jnp.square(h - mean), axis=0, keepdims=True)
        inv = lax.rsqrt(var + 1e-5)
        h_scr[...] = (h - mean) * inv * g_ref[...] + b_ref[...]

    a = h_scr[pl.ds(i * MM_BLK, MM_BLK), :]
    o_ref[...] = lax.dot_general(a, h_scr[...], (((1,), (1,)), ((), ())),
                                 preferred_element_type=jnp.float32)


def _bn_gram(partials, gamma, beta):
    return pl.pallas_call(
        _bn_gram_body,
        grid=(N // MM_BLK,),
        in_specs=[
            pl.BlockSpec((NC, N, HD1), lambda i: (0, 0, 0)),
            pl.BlockSpec((1, HD1), lambda i: (0, 0)),
            pl.BlockSpec((1, HD1), lambda i: (0, 0)),
        ],
        out_specs=pl.BlockSpec((MM_BLK, N), lambda i: (i, 0)),
        out_shape=jax.ShapeDtypeStruct((N, N), jnp.float32),
        scratch_shapes=[pltpu.VMEM((N, HD1), jnp.float32)],
    )(partials, gamma.reshape(1, HD1), beta.reshape(1, HD1))


def kernel(encoder_layer_2, adj, W_att1, W_att2, W_gcn, bn_gamma, bn_beta):
    support = _attention_support(encoder_layer_2, W_att1, W_att2, W_gcn)
    src = adj[1].astype(jnp.int32).reshape(NW, EPW)
    dst = adj[0].astype(jnp.int32).reshape(NW, NCHUNK, CHUNK)
    zeros = jnp.zeros((N, HD1), jnp.float32)
    partials = _build_segsum()(support, src, dst, zeros)
    return _bn_gram(partials, bn_gamma, bn_beta)
